# baseline (speedup 1.0000x reference)
import functools

import numpy as np
import jax
import jax.numpy as jnp
from jax import lax
from jax.experimental import pallas as pl
from jax.experimental.pallas import tpu as pltpu
from jax.experimental.pallas import tpu_sc as plsc

N = 10000
E = 320000
D = 128
DH = D // 2

_INFO = plsc.get_sparse_core_info()
NC = _INFO.num_cores
NS = _INFO.num_subcores
NW = NC * NS
MICRO = 128
E_PER_W = 10240
E_PAD = NW * E_PER_W
ROWS_PER_W = E_PER_W // MICRO
HALF_ROWS = ROWS_PER_W // 2
ACC_N = 10240
N_PER_TILE = ACC_N // NS
L = 16


def _sc_body(xp_hbm, src_hbm, dst_hbm, zeros_hbm, out_hbm,
             sidx, didx, prows, frows, acc, sem0, sem1):
    i32 = np.int32
    c = lax.axis_index("c")
    s = lax.axis_index("s")
    wid = c * i32(NS) + s

    zrows = ACC_N // NS
    pltpu.sync_copy(zeros_hbm.at[pl.ds(s * i32(zrows), zrows)],
                    acc.at[pl.ds(s * i32(zrows), zrows)])
    plsc.subcore_barrier()

    base_e = wid * i32(E_PER_W)
    base_row = wid * i32(ROWS_PER_W)
    sems = (sem0, sem1)

    def start(k, buf):
        return pltpu.async_copy(
            xp_hbm.at[sidx.at[pl.ds(k * MICRO, MICRO)]], prows.at[buf],
            sems[buf])

    def process(k, buf):
        pltpu.make_async_copy(
            xp_hbm.at[sidx.at[pl.ds(k * MICRO, MICRO)]], prows.at[buf],
            sems[buf]).wait()

        def row_body(r, carry):
            for u in range(2):
                rr = r * 2 + u
                for j in range(DH // L):
                    w = prows.at[buf][rr, pl.ds(j * L, L)]
                    lo, hi = plsc.unpack(plsc.bitcast(w, jnp.bfloat16),
                                         format=plsc.PackFormat.INTERLEAVED)
                    frows[rr, pl.ds(j * L, L)] = lo
                    frows[rr, pl.ds(DH + j * L, L)] = hi
            return carry

        lax.fori_loop(i32(0), i32(MICRO // 2), row_body, i32(0))
        pltpu.sync_copy(frows, acc.at[didx.at[k]], add=True)

    for half in range(2):
        r0 = base_row + i32(half * HALF_ROWS)
        e0 = base_e + i32(half * HALF_ROWS * MICRO)
        pltpu.sync_copy(src_hbm.at[pl.ds(e0, HALF_ROWS * MICRO)], sidx)
        pltpu.sync_copy(dst_hbm.at[pl.ds(r0, HALF_ROWS)], didx)

        start(i32(0), 0)

        def pair_body(t, carry):
            k = t * 2
            start(k + 1, 1)
            process(k, 0)
            start(k + 2, 0)
            process(k + 1, 1)
            return carry

        lax.fori_loop(i32(0), i32(HALF_ROWS // 2 - 1), pair_body, i32(0))
        last = i32(HALF_ROWS - 2)
        start(last + 1, 1)
        process(last, 0)
        process(last + 1, 1)

    plsc.subcore_barrier()

    pltpu.sync_copy(acc.at[pl.ds(s * i32(N_PER_TILE), N_PER_TILE)],
                    out_hbm.at[c].at[pl.ds(s * i32(N_PER_TILE), N_PER_TILE)])


_sc_agg = functools.partial(
    pl.kernel,
    out_type=jax.ShapeDtypeStruct((NC, ACC_N, D), jnp.float32),
    mesh=plsc.VectorSubcoreMesh(core_axis_name="c", subcore_axis_name="s"),
    scratch_types=[
        pltpu.VMEM((HALF_ROWS * MICRO,), jnp.int32),
        pltpu.VMEM((HALF_ROWS, MICRO), jnp.int32),
        pltpu.VMEM((2, MICRO, DH), jnp.int32),
        pltpu.VMEM((MICRO, D), jnp.float32),
        pltpu.VMEM_SHARED((ACC_N, D), jnp.float32),
        pltpu.SemaphoreType.DMA,
        pltpu.SemaphoreType.DMA,
    ],
    compiler_params=pltpu.CompilerParams(use_tc_tiling_on_sc=False,
                                         needs_layout_passes=False),
)(_sc_body)


def _tc_body(x_ref, p_ref, ws_ref, wn_ref, b_ref, hl_ref):
    agg = p_ref[0] + p_ref[1]
    o = (
        jnp.dot(x_ref[...], ws_ref[...], preferred_element_type=jnp.float32)
        + jnp.dot(agg, wn_ref[...], preferred_element_type=jnp.float32)
        + b_ref[...]
    )
    o = o * jax.nn.sigmoid(o)
    w = jax.lax.bitcast_convert_type(o, jnp.int32)
    expman = jax.lax.bitwise_and(w, np.int32(0x7FFFFFFF))
    sign = jax.lax.bitwise_xor(w, expman)
    hi = jax.lax.bitwise_or(
        sign,
        jax.lax.shift_right_logical(expman, np.int32(3))
        + np.int32(896 << 20),
    )
    lo = jax.lax.shift_left(w, np.int32(29))
    hl_ref[...] = jnp.stack([lo, hi], axis=-1).reshape(lo.shape[0], 2 * D)


def _tc_tail(x, parts, W_self, W_nbr, b2d):
    blk = 400
    grid = (N // blk,)
    return pl.pallas_call(
        _tc_body,
        grid=grid,
        in_specs=[
            pl.BlockSpec((blk, D), lambda i: (i, 0)),
            pl.BlockSpec((NC, blk, D), lambda i: (0, i, 0)),
            pl.BlockSpec((D, D), lambda i: (0, 0)),
            pl.BlockSpec((D, D), lambda i: (0, 0)),
            pl.BlockSpec((1, D), lambda i: (0, 0)),
        ],
        out_specs=pl.BlockSpec((blk, 2 * D), lambda i: (i, 0)),
        out_shape=jax.ShapeDtypeStruct((N, 2 * D), jnp.int32),
    )(x, parts, W_self, W_nbr, b2d)


@jax.jit
def kernel(x, edge_index, W_self, W_nbr, b):
    with jax.enable_x64(False):
        hl = _impl(x, edge_index, W_self, W_nbr, b)
    return jax.lax.bitcast_convert_type(hl.reshape(N, D, 2), jnp.float64)


def _impl(x, edge_index, W_self, W_nbr, b):
    x = x.astype(jnp.float32)
    W_self = W_self.astype(jnp.float32)
    W_nbr = W_nbr.astype(jnp.float32)
    b = b.astype(jnp.float32)
    src = edge_index[0].astype(jnp.int32)
    dst = edge_index[1].astype(jnp.int32)
    pad = E_PAD - E
    src = jnp.concatenate([src, jnp.zeros((pad,), jnp.int32)])
    dst = jnp.concatenate([dst, jnp.full((pad,), N, jnp.int32)])
    dst2d = dst.reshape(NW * ROWS_PER_W, MICRO)
    zeros = jnp.zeros((ACC_N, D), jnp.float32)
    xb = x.astype(jnp.bfloat16)
    xp = jax.lax.bitcast_convert_type(
        jnp.stack([xb[:, :DH], xb[:, DH:]], axis=-1), jnp.int32)
    parts = _sc_agg(xp, src, dst2d, zeros)
    return _tc_tail(x, parts, W_self, W_nbr, b.reshape(1, D))


# speedup vs baseline: 2.6248x; 2.6248x over previous
import functools

import numpy as np
import jax
import jax.numpy as jnp
from jax import lax
from jax.experimental import pallas as pl
from jax.experimental.pallas import tpu as pltpu
from jax.experimental.pallas import tpu_sc as plsc

N = 10000
E = 320000
D = 128
DH = D // 2

_INFO = plsc.get_sparse_core_info()
NC = _INFO.num_cores
NS = _INFO.num_subcores
NW = NC * NS
MICRO = 128
E_PER_W = 10240
E_PAD = NW * E_PER_W
ROWS_PER_W = E_PER_W // MICRO
HALF_ROWS = ROWS_PER_W // 2
ACC_N = 10240
N_PER_TILE = ACC_N // NS
L = 16


def _sc_body(xp_hbm, src_hbm, dst_hbm, zeros_hbm, out_hbm,
             sidx, didx, prows, frows, acc, sem0, sem1):
    i32 = np.int32
    c = lax.axis_index("c")
    s = lax.axis_index("s")
    wid = c * i32(NS) + s

    zrows = ACC_N // NS
    pltpu.sync_copy(zeros_hbm.at[pl.ds(s * i32(zrows), zrows)],
                    acc.at[pl.ds(s * i32(zrows), zrows)])
    plsc.subcore_barrier()

    base_e = wid * i32(E_PER_W)
    base_row = wid * i32(ROWS_PER_W)
    sems = (sem0, sem1)

    def start(k, buf):
        return pltpu.async_copy(
            xp_hbm.at[sidx.at[pl.ds(k * MICRO, MICRO)]], prows.at[buf],
            sems[buf])

    def process(k, buf):
        pltpu.make_async_copy(
            xp_hbm.at[sidx.at[pl.ds(k * MICRO, MICRO)]], prows.at[buf],
            sems[buf]).wait()

        def row_body(r, carry):
            for u in range(2):
                rr = r * 2 + u
                for j in range(DH // L):
                    w = prows.at[buf][rr, pl.ds(j * L, L)]
                    lo, hi = plsc.unpack(plsc.bitcast(w, jnp.bfloat16),
                                         format=plsc.PackFormat.INTERLEAVED)
                    frows[rr, pl.ds(j * L, L)] = lo
                    frows[rr, pl.ds(DH + j * L, L)] = hi
            return carry

        lax.fori_loop(i32(0), i32(MICRO // 2), row_body, i32(0))
        pltpu.sync_copy(frows, acc.at[didx.at[k]], add=True)

    for half in range(2):
        r0 = base_row + i32(half * HALF_ROWS)
        e0 = base_e + i32(half * HALF_ROWS * MICRO)
        pltpu.sync_copy(src_hbm.at[pl.ds(e0, HALF_ROWS * MICRO)], sidx)
        pltpu.sync_copy(dst_hbm.at[pl.ds(r0, HALF_ROWS)], didx)

        start(i32(0), 0)

        def pair_body(t, carry):
            k = t * 2
            start(k + 1, 1)
            process(k, 0)
            start(k + 2, 0)
            process(k + 1, 1)
            return carry

        lax.fori_loop(i32(0), i32(HALF_ROWS // 2 - 1), pair_body, i32(0))
        last = i32(HALF_ROWS - 2)
        start(last + 1, 1)
        process(last, 0)
        process(last + 1, 1)

    plsc.subcore_barrier()

    pltpu.sync_copy(acc.at[pl.ds(s * i32(N_PER_TILE), N_PER_TILE)],
                    out_hbm.at[c].at[pl.ds(s * i32(N_PER_TILE), N_PER_TILE)])


_sc_agg = functools.partial(
    pl.kernel,
    out_type=jax.ShapeDtypeStruct((NC, ACC_N, D), jnp.float32),
    mesh=plsc.VectorSubcoreMesh(core_axis_name="c", subcore_axis_name="s"),
    scratch_types=[
        pltpu.VMEM((HALF_ROWS * MICRO,), jnp.int32),
        pltpu.VMEM((HALF_ROWS, MICRO), jnp.int32),
        pltpu.VMEM((2, MICRO, DH), jnp.int32),
        pltpu.VMEM((MICRO, D), jnp.float32),
        pltpu.VMEM_SHARED((ACC_N, D), jnp.float32),
        pltpu.SemaphoreType.DMA,
        pltpu.SemaphoreType.DMA,
    ],
    compiler_params=pltpu.CompilerParams(use_tc_tiling_on_sc=False,
                                         needs_layout_passes=False),
)(_sc_body)


def _tc_body(x_ref, p_ref, ws_ref, wn_ref, b_ref, o_ref):
    agg = p_ref[0] + p_ref[1]
    o = (
        jnp.dot(x_ref[...], ws_ref[...], preferred_element_type=jnp.float32)
        + jnp.dot(agg, wn_ref[...], preferred_element_type=jnp.float32)
        + b_ref[...]
    )
    o_ref[...] = o * jax.nn.sigmoid(o)


def _tc_tail(x, parts, W_self, W_nbr, b2d):
    blk = 1000
    grid = (N // blk,)
    return pl.pallas_call(
        _tc_body,
        grid=grid,
        in_specs=[
            pl.BlockSpec((blk, D), lambda i: (i, 0)),
            pl.BlockSpec((NC, blk, D), lambda i: (0, i, 0)),
            pl.BlockSpec((D, D), lambda i: (0, 0)),
            pl.BlockSpec((D, D), lambda i: (0, 0)),
            pl.BlockSpec((1, D), lambda i: (0, 0)),
        ],
        out_specs=pl.BlockSpec((blk, D), lambda i: (i, 0)),
        out_shape=jax.ShapeDtypeStruct((N, D), jnp.float32),
    )(x, parts, W_self, W_nbr, b2d)


@jax.jit
def kernel(x, edge_index, W_self, W_nbr, b):
    out_dtype = jnp.result_type(x.dtype, W_self.dtype)
    with jax.enable_x64(False):
        out = _impl(x, edge_index, W_self, W_nbr, b)
    return out.astype(out_dtype)


def _impl(x, edge_index, W_self, W_nbr, b):
    x = x.astype(jnp.float32)
    W_self = W_self.astype(jnp.float32)
    W_nbr = W_nbr.astype(jnp.float32)
    b = b.astype(jnp.float32)
    src = edge_index[0].astype(jnp.int32)
    dst = edge_index[1].astype(jnp.int32)
    pad = E_PAD - E
    src = jnp.concatenate([src, jnp.zeros((pad,), jnp.int32)])
    dst = jnp.concatenate([dst, jnp.full((pad,), N, jnp.int32)])
    dst2d = dst.reshape(NW * ROWS_PER_W, MICRO)
    zeros = jnp.zeros((ACC_N, D), jnp.float32)
    xb = x.astype(jnp.bfloat16)
    xp = jax.lax.bitcast_convert_type(
        jnp.stack([xb[:, :DH], xb[:, DH:]], axis=-1), jnp.int32)
    parts = _sc_agg(xp, src, dst2d, zeros)
    return _tc_tail(x, parts, W_self, W_nbr, b.reshape(1, D))
